# baseline (device time: 23256 ns/iter reference)
import jax
import jax.numpy as jnp
from jax import lax
from jax.experimental import pallas as pl
from jax.experimental.pallas import tpu as pltpu

N_DEV = 32
N_PLANE = 4
PLANE_SZ = 8
PAIRS = N_DEV // 2
KB = 4


def kernel(x, w_mat):
    m, k = x.shape
    n = w_mat.shape[1]
    nb = n // N_DEV
    gcols = n // N_PLANE
    krows = k // KB

    def body(x_ref, w_hbm, out_ref,
             w_buf, y_ref, recv1, s2_send, s2_recv,
             w_sems, s1_send_sems, s1_recv_sems, s2_send_sems, s2_recv_sems):
        my = lax.axis_index("i")
        p = my // PLANE_SZ
        q = lax.rem(my, PLANE_SZ)

        barrier_sem = pltpu.get_barrier_semaphore()
        for d in range(1, N_PLANE):
            pt = lax.rem(p + d, N_PLANE)
            pl.semaphore_signal(barrier_sem, inc=1,
                                device_id=(pt * PLANE_SZ + q,),
                                device_id_type=pl.DeviceIdType.MESH)
        for d in range(1, PLANE_SZ):
            qt = lax.rem(q + d, PLANE_SZ)
            pl.semaphore_signal(barrier_sem, inc=1,
                                device_id=(p * PLANE_SZ + qt,),
                                device_id_type=pl.DeviceIdType.MESH)

        dmas = []
        for b in range(KB):
            dma = pltpu.make_async_copy(
                w_hbm.at[pl.ds(b * krows, krows), :],
                w_buf.at[b],
                w_sems.at[b],
            )
            dma.start()
            dmas.append(dma)

        xv = x_ref[:, :]
        c = 0.7978845608028654

        acc = None
        for b in range(KB - 1):
            dmas[b].wait()
            part = jnp.dot(xv[:, b * krows:(b + 1) * krows], w_buf[b],
                           preferred_element_type=jnp.float32)
            acc = part if acc is None else acc + part

        dmas[KB - 1].wait()
        pl.semaphore_wait(barrier_sem, N_PLANE - 1 + PLANE_SZ - 1)
        xl = xv[:, (KB - 1) * krows:]
        for a in range(N_PLANE):
            wl = w_buf[KB - 1, :, a * gcols:(a + 1) * gcols]
            blk = acc[:, a * gcols:(a + 1) * gcols] + jnp.dot(
                xl, wl, preferred_element_type=jnp.float32)
            blk = 0.5 * blk * (1.0 + jnp.tanh(c * (blk + 0.044715 * blk * blk * blk)))
            bb = blk.astype(jnp.bfloat16)
            for j in range(4):
                y_ref[a * 4 + j] = bb[:, j * 128:(j + 1) * 128]

            @pl.when(a != p)
            def _():
                rdma = pltpu.make_async_remote_copy(
                    src_ref=y_ref.at[pl.ds(a * 4, 4)],
                    dst_ref=recv1.at[p],
                    send_sem=s1_send_sems.at[a],
                    recv_sem=s1_recv_sems.at[p],
                    device_id=(a * PLANE_SZ + q,),
                    device_id_type=pl.DeviceIdType.MESH,
                )
                rdma.start()

        recv1[pl.ds(p, 1)] = y_ref[pl.ds(p * 4, 4)].reshape(1, 4, m, 128)

        for sp in range(N_PLANE):
            @pl.when(sp != p)
            def _():
                recv = pltpu.make_async_remote_copy(
                    src_ref=y_ref.at[pl.ds(0, 4)],
                    dst_ref=recv1.at[sp],
                    send_sem=s1_send_sems.at[0],
                    recv_sem=s1_recv_sems.at[sp],
                    device_id=(0,),
                    device_id_type=pl.DeviceIdType.MESH,
                )
                recv.wait_recv()

        for mq in range(PLANE_SZ):
            tj, th = mq // 2, mq % 2

            @pl.when(mq != q)
            def _():
                for jj in range(2):
                    va = recv1.at[2 * jj][tj]
                    vb = recv1.at[2 * jj + 1][tj]
                    ca = va[:, th * nb:(th + 1) * nb]
                    cb = vb[:, th * nb:(th + 1) * nb]
                    s2_send[mq, jj] = jnp.concatenate([ca, cb], axis=1)
                rdma = pltpu.make_async_remote_copy(
                    src_ref=s2_send.at[mq],
                    dst_ref=s2_recv.at[q],
                    send_sem=s2_send_sems.at[mq],
                    recv_sem=s2_recv_sems.at[q],
                    device_id=(p * PLANE_SZ + mq,),
                    device_id_type=pl.DeviceIdType.MESH,
                )
                rdma.start()

        tjq = q // 2
        thq = lax.rem(q, 2)
        for pp in range(N_PLANE):
            v = recv1.at[pp][pl.ds(tjq, 1)].reshape(m, 128)
            val = jnp.where(thq == 0, v[:, :nb], v[:, nb:])
            out_ref[pl.ds((pp * PLANE_SZ + q) * m, m), :] = val.astype(jnp.float32)

        for sq in range(PLANE_SZ):
            @pl.when(sq != q)
            def _():
                recv = pltpu.make_async_remote_copy(
                    src_ref=s2_send.at[0],
                    dst_ref=s2_recv.at[sq],
                    send_sem=s2_send_sems.at[0],
                    recv_sem=s2_recv_sems.at[sq],
                    device_id=(0,),
                    device_id_type=pl.DeviceIdType.MESH,
                )
                recv.wait_recv()
                for jj in range(2):
                    v = s2_recv.at[sq][jj]
                    for dd in range(2):
                        src_rank = (2 * jj + dd) * PLANE_SZ + sq
                        val = v[:, dd * nb:(dd + 1) * nb]
                        out_ref[pl.ds(src_rank * m, m), :] = val.astype(jnp.float32)

        for a in range(N_PLANE):
            @pl.when(a != p)
            def _():
                dr = pltpu.make_async_remote_copy(
                    src_ref=y_ref.at[pl.ds(0, 4)],
                    dst_ref=recv1.at[0],
                    send_sem=s1_send_sems.at[a],
                    recv_sem=s1_recv_sems.at[0],
                    device_id=(0,),
                    device_id_type=pl.DeviceIdType.MESH,
                )
                dr.wait_send()
        for mq in range(PLANE_SZ):
            @pl.when(mq != q)
            def _():
                dr = pltpu.make_async_remote_copy(
                    src_ref=s2_send.at[mq],
                    dst_ref=s2_recv.at[0],
                    send_sem=s2_send_sems.at[mq],
                    recv_sem=s2_recv_sems.at[0],
                    device_id=(0,),
                    device_id_type=pl.DeviceIdType.MESH,
                )
                dr.wait_send()

    return pl.pallas_call(
        body,
        out_shape=jax.ShapeDtypeStruct((N_DEV * m, nb), jnp.float32),
        in_specs=[
            pl.BlockSpec(memory_space=pltpu.VMEM),
            pl.BlockSpec(memory_space=pl.ANY),
        ],
        out_specs=pl.BlockSpec(memory_space=pltpu.VMEM),
        scratch_shapes=[
            pltpu.VMEM((KB, krows, n), jnp.float32),
            pltpu.VMEM((PAIRS, m, 128), jnp.bfloat16),
            pltpu.VMEM((N_PLANE, 4, m, 128), jnp.bfloat16),
            pltpu.VMEM((PLANE_SZ, 2, m, 128), jnp.bfloat16),
            pltpu.VMEM((PLANE_SZ, 2, m, 128), jnp.bfloat16),
            pltpu.SemaphoreType.DMA((KB,)),
            pltpu.SemaphoreType.DMA((N_PLANE,)),
            pltpu.SemaphoreType.DMA((N_PLANE,)),
            pltpu.SemaphoreType.DMA((PLANE_SZ,)),
            pltpu.SemaphoreType.DMA((PLANE_SZ,)),
        ],
        compiler_params=pltpu.CompilerParams(collective_id=0),
    )(x, w_mat)
